# baseline (speedup 1.0000x reference)
import functools

import jax
import jax.numpy as jnp
from jax import lax
from jax.experimental import pallas as pl
from jax.experimental.pallas import tpu as pltpu
from jax.experimental.pallas import tpu_sc as plsc

_NC, _NS = 2, 16
_G = 128
_TR = 160
_CHS = _NS * _TR
_PER_TILE = 5


def kernel(x, table):
    B, S = x.shape
    V, D = table.shape
    R = B * S * D // 128
    r_sc = R // _NC
    assert r_sc == _CHS * _PER_TILE * _NS

    mesh = plsc.VectorSubcoreMesh(core_axis_name="c", subcore_axis_name="s")

    @functools.partial(
        pl.kernel,
        mesh=mesh,
        out_type=jax.ShapeDtypeStruct((R, 128), jnp.float32),
        scratch_types=[
            pltpu.VMEM((_TR, 128), jnp.float32),
            pltpu.VMEM((_G,), jnp.int32),
            pltpu.VMEM_SHARED((_CHS, 128), jnp.float32),
            pltpu.SemaphoreType.DMA,
            pltpu.SemaphoreType.DMA,
        ],
    )
    def sc_fill(table_hbm, out_hbm, buf, idx, shared, sem_g, sem_o):
        cid = lax.axis_index("c")
        sid = lax.axis_index("s")

        for i in range(_G // 16):
            idx[pl.ds(i * 16, 16)] = jnp.zeros((16,), jnp.int32)

        pltpu.async_copy(table_hbm.at[idx], buf.at[pl.ds(0, _G)], sem_g).wait()

        vregs = [buf[0, pl.ds(16 * j, 16)] for j in range(8)]

        def rep(i, _):
            for j in range(8):
                buf[i, pl.ds(16 * j, 16)] = vregs[j]
            return 0

        lax.fori_loop(_G, _TR, rep, 0)

        pltpu.sync_copy(buf, shared.at[pl.ds(sid * _TR, _TR)])
        plsc.subcore_barrier()

        copies = [
            pltpu.async_copy(
                shared,
                out_hbm.at[pl.ds(cid * r_sc + (sid * _PER_TILE + k) * _CHS, _CHS)],
                sem_o,
            )
            for k in range(_PER_TILE)
        ]
        for cp in copies:
            cp.wait()

    out = sc_fill(jnp.tile(table, (1, 2)))
    return out.reshape(B, S, D)
